# baseline (speedup 1.0000x reference)
import functools

import jax
import jax.numpy as jnp
from jax.experimental import pallas as pl
from jax.experimental.pallas import tpu as pltpu

N_ATOM = 10000
E_ATOM = 320000
E_LINE = 500000
HID = 128
NGRAPH = 64
NLAYERS = 2

_BR = 2000


def _mm_body(x_ref, w_ref, b_ref, out_ref):
    out_ref[...] = (
        jnp.dot(x_ref[...], w_ref[...], preferred_element_type=jnp.float32)
        + b_ref[...]
    )


def _mm(x, w, b):
    r, k = x.shape
    f = w.shape[1]
    return pl.pallas_call(
        _mm_body,
        grid=(r // _BR,),
        in_specs=[
            pl.BlockSpec((_BR, k), lambda i: (i, 0)),
            pl.BlockSpec((k, f), lambda i: (0, 0)),
            pl.BlockSpec((1, f), lambda i: (0, 0)),
        ],
        out_specs=pl.BlockSpec((_BR, f), lambda i: (i, 0)),
        out_shape=jax.ShapeDtypeStruct((r, f), jnp.float32),
    )(x, w, b[None, :])


def _mlp_body(pooled_ref, w1_ref, b1_ref, w2_ref, b2_ref, out_ref):
    hid = jnp.maximum(pooled_ref[...] @ w1_ref[...] + b1_ref[...], 0.0)
    out_ref[...] = hid @ w2_ref[...] + b2_ref[...]


def _final_mlp(pooled, w1, b1, w2, b2):
    return pl.pallas_call(
        _mlp_body,
        out_shape=jax.ShapeDtypeStruct((NGRAPH, w2.shape[1]), jnp.float32),
    )(pooled, w1, b1[None, :], w2, b2[None, :])


def _layer(x, e, src, dst, nW, nb, eW, eb, gW, gb, num_nodes):
    w_src = jnp.concatenate([nW[:HID], eW[:HID]], axis=1)
    w_dst = jnp.concatenate([gW[:HID], eW[HID:2 * HID]], axis=1)
    w_edge = jnp.concatenate([nW[HID:], gW[HID:], eW[2 * HID:]], axis=1)
    b_edge = jnp.concatenate([nb, gb, eb])

    t_src = _mm(x, w_src, jnp.zeros((2 * HID,), jnp.float32))
    t_dst = _mm(x, w_dst, jnp.zeros((2 * HID,), jnp.float32))
    u = _mm(e, w_edge, b_edge)

    g_src = t_src[src]
    g_dst = t_dst[dst]
    gate = jax.nn.sigmoid(g_dst[:, :HID] + u[:, HID:2 * HID])
    msg = gate * (g_src[:, :HID] + u[:, :HID])
    e_new = e + g_src[:, HID:] + g_dst[:, HID:] + u[:, 2 * HID:]
    x_new = x + jax.ops.segment_sum(msg, dst, num_segments=num_nodes)
    return x_new, e_new


def kernel(atom_x, atom_edge_index, atom_edge_attr, atom_batch, line_x,
           line_edge_index, line_edge_attr, node_embed_W, node_embed_b,
           edge_embed_W, edge_embed_b, line_edge_embed_W, line_edge_embed_b,
           atom_node_W, atom_node_b, atom_edgemlp_W, atom_edgemlp_b,
           atom_gate_W, atom_gate_b, line_node_W, line_node_b,
           line_edgemlp_W, line_edgemlp_b, line_gate_W, line_gate_b,
           mlp_W1, mlp_b1, mlp_W2, mlp_b2):
    h = _mm(atom_x, node_embed_W, node_embed_b)
    e = _mm(atom_edge_attr, edge_embed_W, edge_embed_b)
    l = _mm(line_x, edge_embed_W, edge_embed_b)
    le = _mm(line_edge_attr, line_edge_embed_W, line_edge_embed_b)

    for k in range(NLAYERS):
        l, le = _layer(l, le, line_edge_index[0], line_edge_index[1],
                       line_node_W[k], line_node_b[k], line_edgemlp_W[k],
                       line_edgemlp_b[k], line_gate_W[k], line_gate_b[k],
                       E_ATOM)
        h, e = _layer(h, e, atom_edge_index[0], atom_edge_index[1],
                      atom_node_W[k], atom_node_b[k], atom_edgemlp_W[k],
                      atom_edgemlp_b[k], atom_gate_W[k], atom_gate_b[k],
                      N_ATOM)

    sums = jax.ops.segment_sum(h, atom_batch, num_segments=NGRAPH)
    cnt = jax.ops.segment_sum(jnp.ones((h.shape[0], 1), jnp.float32),
                              atom_batch, num_segments=NGRAPH)
    pooled = sums / jnp.maximum(cnt, 1.0)
    return _final_mlp(pooled, mlp_W1, mlp_b1, mlp_W2, mlp_b2)


# speedup vs baseline: 1.0001x; 1.0001x over previous
import functools

import jax
import jax.numpy as jnp
from jax.experimental import pallas as pl
from jax.experimental.pallas import tpu as pltpu

N_ATOM = 10000
E_ATOM = 320000
E_LINE = 500000
HID = 128
NGRAPH = 64
NLAYERS = 2

_BR = 2000


def _mm_body(x_ref, w_ref, b_ref, out_ref):
    out_ref[...] = (
        jnp.dot(x_ref[...], w_ref[...], preferred_element_type=jnp.float32)
        + b_ref[...]
    )


def _mm(x, w, b):
    r, k = x.shape
    f = w.shape[1]
    return pl.pallas_call(
        _mm_body,
        grid=(r // _BR,),
        in_specs=[
            pl.BlockSpec((_BR, k), lambda i: (i, 0)),
            pl.BlockSpec((k, f), lambda i: (0, 0)),
            pl.BlockSpec((1, f), lambda i: (0, 0)),
        ],
        out_specs=pl.BlockSpec((_BR, f), lambda i: (i, 0)),
        out_shape=jax.ShapeDtypeStruct((r, f), jnp.float32),
    )(x, w, b[None, :])


def _mlp_body(pooled_ref, w1_ref, b1_ref, w2_ref, b2_ref, out_ref):
    hid = jnp.maximum(pooled_ref[...] @ w1_ref[...] + b1_ref[...], 0.0)
    out_ref[...] = hid @ w2_ref[...] + b2_ref[...]


def _final_mlp(pooled, w1, b1, w2, b2):
    return pl.pallas_call(
        _mlp_body,
        out_shape=jax.ShapeDtypeStruct((NGRAPH, w2.shape[1]), jnp.float32),
    )(pooled, w1, b1[None, :], w2, b2[None, :])


def _layer(x, e, src, dst, nW, nb, eW, eb, gW, gb, num_nodes):
    w_src = jnp.concatenate([nW[:HID], eW[:HID]], axis=1)
    w_dst = jnp.concatenate([gW[:HID], eW[HID:2 * HID]], axis=1)
    w_edge = jnp.concatenate([nW[HID:], gW[HID:], eW[2 * HID:]], axis=1)
    b_edge = jnp.concatenate([nb, gb, eb])

    t_src = _mm(x, w_src, jnp.zeros((2 * HID,), jnp.float32))
    t_dst = _mm(x, w_dst, jnp.zeros((2 * HID,), jnp.float32))
    u = _mm(e, w_edge, b_edge)

    g_src = t_src[src]
    g_dst = t_dst[dst]
    gate = jax.nn.sigmoid(g_dst[:, :HID] + u[:, HID:2 * HID])
    msg = gate * (g_src[:, :HID] + u[:, :HID])
    e_new = e + g_src[:, HID:] + g_dst[:, HID:] + u[:, 2 * HID:]
    x_new = x + jax.ops.segment_sum(msg, dst, num_segments=num_nodes)
    return x_new, e_new


def kernel(atom_x, atom_edge_index, atom_edge_attr, atom_batch, line_x,
           line_edge_index, line_edge_attr, node_embed_W, node_embed_b,
           edge_embed_W, edge_embed_b, line_edge_embed_W, line_edge_embed_b,
           atom_node_W, atom_node_b, atom_edgemlp_W, atom_edgemlp_b,
           atom_gate_W, atom_gate_b, line_node_W, line_node_b,
           line_edgemlp_W, line_edgemlp_b, line_gate_W, line_gate_b,
           mlp_W1, mlp_b1, mlp_W2, mlp_b2):
    h = _mm(atom_x, node_embed_W, node_embed_b)
    e = _mm(atom_edge_attr, edge_embed_W, edge_embed_b)

    for k in range(NLAYERS):
        h, e = _layer(h, e, atom_edge_index[0], atom_edge_index[1],
                      atom_node_W[k], atom_node_b[k], atom_edgemlp_W[k],
                      atom_edgemlp_b[k], atom_gate_W[k], atom_gate_b[k],
                      N_ATOM)

    sums = jax.ops.segment_sum(h, atom_batch, num_segments=NGRAPH)
    cnt = jax.ops.segment_sum(jnp.ones((h.shape[0], 1), jnp.float32),
                              atom_batch, num_segments=NGRAPH)
    pooled = sums / jnp.maximum(cnt, 1.0)
    return _final_mlp(pooled, mlp_W1, mlp_b1, mlp_W2, mlp_b2)
